# baseline (speedup 1.0000x reference)
import functools

import jax
import jax.numpy as jnp
from jax import lax
from jax.experimental import pallas as pl
from jax.experimental.pallas import tpu as pltpu
from jax.experimental.pallas import tpu_sc as plsc


def _build_gather_t(B, V, D):
    info = plsc.get_sparse_core_info()
    NC, NS = info.num_cores, info.num_subcores
    NW = NC * NS
    assert B % NW == 0
    b_per_w = B // NW
    mesh = plsc.VectorSubcoreMesh(core_axis_name="c", subcore_axis_name="s")

    @functools.partial(
        pl.kernel,
        mesh=mesh,
        out_type=jax.ShapeDtypeStruct((D, B), jnp.float32),
        scratch_types=[
            pltpu.VMEM((b_per_w,), jnp.int32),
            pltpu.VMEM((D, b_per_w), jnp.float32),
            pltpu.SemaphoreType.DMA,
        ],
        compiler_params=pltpu.CompilerParams(use_tc_tiling_on_sc=False),
    )
    def gather_kernel(ids_hbm, table_t_hbm, out_t_hbm, idx_v, rows_v, sem):
        wid = lax.axis_index("s") * NC + lax.axis_index("c")
        base = wid * b_per_w
        pltpu.sync_copy(ids_hbm.at[pl.ds(base, b_per_w)], idx_v)
        copies = [
            pltpu.async_copy(table_t_hbm.at[d].at[idx_v], rows_v.at[d], sem)
            for d in range(D)
        ]
        for c in copies:
            c.wait()
        pltpu.sync_copy(rows_v, out_t_hbm.at[:, pl.ds(base, b_per_w)])

    return gather_kernel


def kernel(intent_ids, embedding_table):
    if intent_ids.ndim == 2:
        intent_ids = jnp.squeeze(intent_ids, axis=1)
    ids = intent_ids.astype(jnp.int32)
    B = ids.shape[0]
    V, D = embedding_table.shape
    out_t = _build_gather_t(B, V, D)(ids, embedding_table.T)
    return out_t.T
